# Initial kernel scaffold; baseline (speedup 1.0000x reference)
import jax
import jax.numpy as jnp
from jax.experimental import pallas as pl


def kernel(features, edge_index, W1, b1, W2, b2, fc_w, fc_b):
    raise NotImplementedError("write your pallas kernel here")


# speedup vs baseline: 5.6317x; 5.6317x over previous
import functools

import jax
import jax.numpy as jnp
from jax import lax
from jax.experimental import pallas as pl
from jax.experimental.pallas import tpu as pltpu
from jax.experimental.pallas import tpu_sc as plsc

N = 286
NPAD = 288
E = 9152
EPAD = 9216
NCH_N = NPAD // 16
NCH_E = EPAD // 16

OFF_F0 = 0
OFF_F1 = NPAD
OFF_W0 = 2 * NPAD
OFF_W1 = 3 * NPAD
OFF_P = 4 * NPAD


def _rsqrt16(x):
    i = plsc.bitcast(x, jnp.int32)
    y = plsc.bitcast(jnp.int32(0x5F3759DF) - lax.shift_right_arithmetic(i, 1),
                     jnp.float32)
    for _ in range(3):
        y = y * (1.5 - 0.5 * x * y * y)
    return y


def _gnn_body(edges_hbm, consts_hbm, out_hbm,
              src_v, dst_v, consts_v, x0, x1, h0, h1, a0, a1,
              onorm, inorm, res_v):
    is_lead = jnp.logical_and(lax.axis_index("c") == 0,
                              lax.axis_index("s") == 0)

    @pl.when(is_lead)
    def _():
        pltpu.sync_copy(edges_hbm.at[0], src_v)
        pltpu.sync_copy(edges_hbm.at[1], dst_v)
        pltpu.sync_copy(consts_hbm, consts_v)

        zeros16 = jnp.zeros((16,), jnp.float32)
        ones16 = jnp.ones((16,), jnp.float32)

        def bc(j):
            return plsc.load_gather(
                consts_v, [jnp.full((16,), OFF_P + j, jnp.int32)])

        def zero_deg(i, c):
            sl = pl.ds(i * 16, 16)
            onorm[sl] = zeros16
            inorm[sl] = zeros16
            return c
        lax.fori_loop(0, NCH_N, zero_deg, 0)

        def deg(i, c):
            sl = pl.ds(i * 16, 16)
            plsc.addupdate_scatter(onorm, [src_v[sl]], ones16)
            plsc.addupdate_scatter(inorm, [dst_v[sl]], ones16)
            return c
        lax.fori_loop(0, NCH_E, deg, 0)

        def norm(i, c):
            sl = pl.ds(i * 16, 16)
            onorm[sl] = _rsqrt16(jnp.maximum(onorm[sl], 1.0))
            inorm[sl] = _rsqrt16(jnp.maximum(inorm[sl], 1.0))
            return c
        lax.fori_loop(0, NCH_N, norm, 0)

        def layer(get0, get1, pbase, wr0, wr1):
            w00, w01, w10, w11 = bc(pbase), bc(pbase + 1), bc(pbase + 2), bc(pbase + 3)
            b0, b1 = bc(pbase + 4), bc(pbase + 5)

            def pre(i, c):
                sl = pl.ds(i * 16, 16)
                on = onorm[sl]
                v0 = get0(i) * on
                v1 = get1(i) * on
                h0[sl] = v0 * w00 + v1 * w10
                h1[sl] = v0 * w01 + v1 * w11
                a0[sl] = zeros16
                a1[sl] = zeros16
                return c
            lax.fori_loop(0, NCH_N, pre, 0)

            def edge(i, c):
                sl = pl.ds(i * 16, 16)
                s = src_v[sl]
                d = dst_v[sl]
                plsc.addupdate_scatter(a0, [d], plsc.load_gather(h0, [s]))
                plsc.addupdate_scatter(a1, [d], plsc.load_gather(h1, [s]))
                return c
            lax.fori_loop(0, NCH_E, edge, 0)

            def post(i, c):
                sl = pl.ds(i * 16, 16)
                inn = inorm[sl]
                wr0[sl] = jnp.maximum(a0[sl] * inn + b0, 0.0)
                wr1[sl] = jnp.maximum(a1[sl] * inn + b1, 0.0)
                return c
            lax.fori_loop(0, NCH_N, post, 0)

        layer(lambda i: consts_v[pl.ds(OFF_F0 + i * 16, 16)],
              lambda i: consts_v[pl.ds(OFF_F1 + i * 16, 16)],
              0, x0, x1)
        layer(lambda i: x0[pl.ds(i * 16, 16)],
              lambda i: x1[pl.ds(i * 16, 16)],
              6, x0, x1)

        def fc(i, acc):
            sl = pl.ds(i * 16, 16)
            return (acc + x0[sl] * consts_v[pl.ds(OFF_W0 + i * 16, 16)]
                    + x1[sl] * consts_v[pl.ds(OFF_W1 + i * 16, 16)])
        acc = lax.fori_loop(0, NCH_N, fc, zeros16)
        tot = jnp.full((16,), jnp.sum(acc)) + bc(12)
        res_v[...] = 1.0 / (1.0 + jnp.exp(-tot))
        pltpu.sync_copy(res_v, out_hbm)


_gnn = functools.partial(
    pl.kernel,
    out_type=jax.ShapeDtypeStruct((16,), jnp.float32),
    mesh=plsc.VectorSubcoreMesh(core_axis_name="c", subcore_axis_name="s",
                                num_cores=2, num_subcores=16),
    compiler_params=pltpu.CompilerParams(needs_layout_passes=False),
    scratch_types=[
        pltpu.VMEM((EPAD,), jnp.int32),
        pltpu.VMEM((EPAD,), jnp.int32),
        pltpu.VMEM((OFF_P + 16,), jnp.float32),
        pltpu.VMEM((NPAD,), jnp.float32),
        pltpu.VMEM((NPAD,), jnp.float32),
        pltpu.VMEM((NPAD,), jnp.float32),
        pltpu.VMEM((NPAD,), jnp.float32),
        pltpu.VMEM((NPAD,), jnp.float32),
        pltpu.VMEM((NPAD,), jnp.float32),
        pltpu.VMEM((NPAD,), jnp.float32),
        pltpu.VMEM((NPAD,), jnp.float32),
        pltpu.VMEM((16,), jnp.float32),
    ],
)(_gnn_body)


def kernel(features, edge_index, W1, b1, W2, b2, fc_w, fc_b):
    pad = jnp.full((2, EPAD - E), N, jnp.int32)
    edges = jnp.concatenate([edge_index, pad], axis=1)

    zpad = jnp.zeros((NPAD - N,), jnp.float32)
    fcw = fc_w[:, 0].reshape(N, 2)
    consts = jnp.concatenate([
        features[:, 0], zpad,
        features[:, 1], zpad,
        fcw[:, 0], zpad,
        fcw[:, 1], zpad,
        W1.ravel(), b1, W2.ravel(), b2, fc_b, jnp.zeros((3,), jnp.float32),
    ])

    res = _gnn(edges, consts)
    return res[0:1].reshape(1, 1)


# speedup vs baseline: 9.3658x; 1.6630x over previous
import functools

import jax
import jax.numpy as jnp
from jax import lax
from jax.experimental import pallas as pl
from jax.experimental.pallas import tpu as pltpu
from jax.experimental.pallas import tpu_sc as plsc

N = 286
NPAD = 288
E = 9152
EPAD = 9216
NCH_N = NPAD // 16
NCH_E = EPAD // 16
NSUB = 16
EPT = EPAD // NSUB
ECH_W = EPT // 16

ACC_COLS = 48
COL_B = 24

OFF_F0 = 0
OFF_F1 = NPAD
OFF_W0 = 2 * NPAD
OFF_W1 = 3 * NPAD
OFF_P = 4 * NPAD


def _rsqrt16(x):
    i = plsc.bitcast(x, jnp.int32)
    y = plsc.bitcast(jnp.int32(0x5F3759DF) - jnp.right_shift(i, 1),
                     jnp.float32)
    for _ in range(3):
        y = y * (1.5 - 0.5 * x * y * y)
    return y


def _gnn_body(src_hbm, dst_hbm, consts_hbm, out_hbm,
              src_v, dst_v, consts_v, x0, x1, h0, h1, onorm, inorm,
              acc, shared, res_v):
    w = lax.axis_index("s")
    is_lead = jnp.logical_and(lax.axis_index("c") == 0, w == 0)

    pltpu.sync_copy(src_hbm.at[pl.ds(w * EPT, EPT)], src_v)
    pltpu.sync_copy(dst_hbm.at[pl.ds(w * EPT, EPT)], dst_v)
    pltpu.sync_copy(consts_hbm, consts_v)

    zeros16 = jnp.zeros((16,), jnp.float32)
    ones16 = jnp.ones((16,), jnp.float32)
    iota16 = lax.iota(jnp.int32, 16)

    def bc(j):
        return plsc.load_gather(
            consts_v, [jnp.full((16,), OFF_P + j, jnp.int32)])

    def zero_acc(r, c):
        acc[r, pl.ds(0, 16)] = zeros16
        acc[r, pl.ds(16, 16)] = zeros16
        acc[r, pl.ds(32, 16)] = zeros16
        return c

    def reduce_acc():
        plsc.subcore_barrier()

        @pl.when(w == 0)
        def _():
            pltpu.sync_copy(acc, shared)
        plsc.subcore_barrier()

        @pl.when(w != 0)
        def _():
            pltpu.sync_copy(acc, shared.at[iota16], add=True)
        plsc.subcore_barrier()
        pltpu.sync_copy(shared, acc)

    lax.fori_loop(0, NSUB, zero_acc, 0)

    def deg(i, c):
        sl = pl.ds(i * 16, 16)
        s = src_v[sl]
        d = dst_v[sl]
        plsc.addupdate_scatter(acc, [s & 15, jnp.right_shift(s, 4)], ones16)
        plsc.addupdate_scatter(
            acc, [d & 15, jnp.right_shift(d, 4) + COL_B], ones16)
        return c
    lax.fori_loop(0, ECH_W, deg, 0)
    reduce_acc()

    def norm(i, c):
        sl = pl.ds(i * 16, 16)
        fi = jnp.full((16,), i, jnp.int32)
        dv0 = plsc.load_gather(acc, [iota16, fi])
        dv1 = plsc.load_gather(acc, [iota16, fi + COL_B])
        onorm[sl] = _rsqrt16(jnp.maximum(dv0, 1.0))
        inorm[sl] = _rsqrt16(jnp.maximum(dv1, 1.0))
        return c
    lax.fori_loop(0, NCH_N, norm, 0)

    def layer(get0, get1, pbase, wr0, wr1):
        w00, w01, w10, w11 = bc(pbase), bc(pbase + 1), bc(pbase + 2), bc(pbase + 3)
        b0, b1 = bc(pbase + 4), bc(pbase + 5)

        def pre(i, c):
            sl = pl.ds(i * 16, 16)
            on = onorm[sl]
            v0 = get0(i) * on
            v1 = get1(i) * on
            h0[sl] = v0 * w00 + v1 * w10
            h1[sl] = v0 * w01 + v1 * w11
            return c
        lax.fori_loop(0, NCH_N, pre, 0)
        lax.fori_loop(0, NSUB, zero_acc, 0)

        def edge(i, c):
            sl = pl.ds(i * 16, 16)
            s = src_v[sl]
            d = dst_v[sl]
            m0 = plsc.load_gather(h0, [s])
            m1 = plsc.load_gather(h1, [s])
            dlo = d & 15
            dhi = jnp.right_shift(d, 4)
            plsc.addupdate_scatter(acc, [dlo, dhi], m0)
            plsc.addupdate_scatter(acc, [dlo, dhi + COL_B], m1)
            return c
        lax.fori_loop(0, ECH_W, edge, 0)
        reduce_acc()

        def post(i, c):
            sl = pl.ds(i * 16, 16)
            inn = inorm[sl]
            fi = jnp.full((16,), i, jnp.int32)
            av0 = plsc.load_gather(acc, [iota16, fi])
            av1 = plsc.load_gather(acc, [iota16, fi + COL_B])
            wr0[sl] = jnp.maximum(av0 * inn + b0, 0.0)
            wr1[sl] = jnp.maximum(av1 * inn + b1, 0.0)
            return c
        lax.fori_loop(0, NCH_N, post, 0)

    layer(lambda i: consts_v[pl.ds(OFF_F0 + i * 16, 16)],
          lambda i: consts_v[pl.ds(OFF_F1 + i * 16, 16)],
          0, x0, x1)
    layer(lambda i: x0[pl.ds(i * 16, 16)],
          lambda i: x1[pl.ds(i * 16, 16)],
          6, x0, x1)

    @pl.when(is_lead)
    def _():
        def fc(i, a):
            sl = pl.ds(i * 16, 16)
            return (a + x0[sl] * consts_v[pl.ds(OFF_W0 + i * 16, 16)]
                    + x1[sl] * consts_v[pl.ds(OFF_W1 + i * 16, 16)])
        a = lax.fori_loop(0, NCH_N, fc, zeros16)
        tot = jnp.full((16,), jnp.sum(a)) + bc(12)
        res_v[...] = 1.0 / (1.0 + jnp.exp(-tot))
        pltpu.sync_copy(res_v, out_hbm)


_gnn = functools.partial(
    pl.kernel,
    out_type=jax.ShapeDtypeStruct((16,), jnp.float32),
    mesh=plsc.VectorSubcoreMesh(core_axis_name="c", subcore_axis_name="s",
                                num_cores=2, num_subcores=16),
    compiler_params=pltpu.CompilerParams(needs_layout_passes=False),
    scratch_types=[
        pltpu.VMEM((EPT,), jnp.int32),
        pltpu.VMEM((EPT,), jnp.int32),
        pltpu.VMEM((OFF_P + 16,), jnp.float32),
        pltpu.VMEM((NPAD,), jnp.float32),
        pltpu.VMEM((NPAD,), jnp.float32),
        pltpu.VMEM((NPAD,), jnp.float32),
        pltpu.VMEM((NPAD,), jnp.float32),
        pltpu.VMEM((NPAD,), jnp.float32),
        pltpu.VMEM((NPAD,), jnp.float32),
        pltpu.VMEM((NSUB, ACC_COLS), jnp.float32),
        pltpu.VMEM_SHARED((NSUB, ACC_COLS), jnp.float32),
        pltpu.VMEM((16,), jnp.float32),
    ],
)(_gnn_body)


def kernel(features, edge_index, W1, b1, W2, b2, fc_w, fc_b):
    pad = jnp.full((EPAD - E,), N, jnp.int32)
    src = jnp.concatenate([edge_index[0], pad])
    dst = jnp.concatenate([edge_index[1], pad])

    zpad = jnp.zeros((NPAD - N,), jnp.float32)
    fcw = fc_w[:, 0].reshape(N, 2)
    consts = jnp.concatenate([
        features[:, 0], zpad,
        features[:, 1], zpad,
        fcw[:, 0], zpad,
        fcw[:, 1], zpad,
        W1.ravel(), b1, W2.ravel(), b2, fc_b, jnp.zeros((3,), jnp.float32),
    ])

    res = _gnn(src, dst, consts)
    return res[0:1].reshape(1, 1)


# speedup vs baseline: 10.0299x; 1.0709x over previous
import functools

import jax
import jax.numpy as jnp
from jax import lax
from jax.experimental import pallas as pl
from jax.experimental.pallas import tpu as pltpu
from jax.experimental.pallas import tpu_sc as plsc

N = 286
NPAD = 288
E = 9152
EPAD = 9216
NCH_N = NPAD // 16
NCH_E = EPAD // 16
NSUB = 16
EPT = EPAD // NSUB
ECH_W = EPT // 16

ACC_COLS = 48
COL_B = 24

OFF_F0 = 0
OFF_F1 = NPAD
OFF_W0 = 2 * NPAD
OFF_W1 = 3 * NPAD
OFF_P = 4 * NPAD


def _rsqrt16(x):
    i = plsc.bitcast(x, jnp.int32)
    y = plsc.bitcast(jnp.int32(0x5F3759DF) - jnp.right_shift(i, 1),
                     jnp.float32)
    for _ in range(3):
        y = y * (1.5 - 0.5 * x * y * y)
    return y


def _gnn_body(src_hbm, dst_hbm, consts_hbm, out_hbm,
              src_v, dst_v, consts_v, x0, x1, h0, h1, onorm, inorm,
              acc, shared, res_v):
    w = lax.axis_index("s")
    is_lead = jnp.logical_and(lax.axis_index("c") == 0, w == 0)

    pltpu.sync_copy(src_hbm.at[pl.ds(w * EPT, EPT)], src_v)
    pltpu.sync_copy(dst_hbm.at[pl.ds(w * EPT, EPT)], dst_v)
    pltpu.sync_copy(consts_hbm, consts_v)

    zeros16 = jnp.zeros((16,), jnp.float32)
    ones16 = jnp.ones((16,), jnp.float32)
    iota16 = lax.iota(jnp.int32, 16)

    def bc(j):
        return plsc.load_gather(
            consts_v, [jnp.full((16,), OFF_P + j, jnp.int32)])

    def zero_acc(r, c):
        acc[r, pl.ds(0, 16)] = zeros16
        acc[r, pl.ds(16, 16)] = zeros16
        acc[r, pl.ds(32, 16)] = zeros16
        return c

    def reduce_acc():
        plsc.subcore_barrier()

        @pl.when(w == 0)
        def _():
            pltpu.sync_copy(acc, shared)
        plsc.subcore_barrier()

        @pl.when(w != 0)
        def _():
            pltpu.sync_copy(acc, shared.at[iota16], add=True)
        plsc.subcore_barrier()
        pltpu.sync_copy(shared, acc)

    lax.fori_loop(0, NSUB, zero_acc, 0)

    def deg(i, c):
        sl = pl.ds(i * 16, 16)
        s = src_v[sl]
        d = dst_v[sl]
        plsc.addupdate_scatter(acc, [s & 15, jnp.right_shift(s, 4)], ones16)
        plsc.addupdate_scatter(
            acc, [d & 15, jnp.right_shift(d, 4) + COL_B], ones16)
        return c
    lax.fori_loop(0, ECH_W, deg, 0)
    reduce_acc()

    def norm(i, c):
        sl = pl.ds(i * 16, 16)
        fi = jnp.full((16,), i, jnp.int32)
        dv0 = plsc.load_gather(acc, [iota16, fi])
        dv1 = plsc.load_gather(acc, [iota16, fi + COL_B])
        onorm[sl] = _rsqrt16(jnp.maximum(dv0, 1.0))
        inorm[sl] = _rsqrt16(jnp.maximum(dv1, 1.0))
        return c
    lax.fori_loop(0, NCH_N, norm, 0)

    def layer(get0, get1, pbase, wr0, wr1):
        w00, w01, w10, w11 = bc(pbase), bc(pbase + 1), bc(pbase + 2), bc(pbase + 3)
        b0, b1 = bc(pbase + 4), bc(pbase + 5)

        def pre(i, c):
            sl = pl.ds(i * 16, 16)
            on = onorm[sl]
            v0 = get0(i) * on
            v1 = get1(i) * on
            h0[sl] = v0 * w00 + v1 * w10
            h1[sl] = v0 * w01 + v1 * w11
            return c
        lax.fori_loop(0, NCH_N, pre, 0)
        lax.fori_loop(0, NSUB, zero_acc, 0)

        def edge(i, c):
            sl = pl.ds(i * 16, 16)
            s = src_v[sl]
            d = dst_v[sl]
            m0 = plsc.load_gather(h0, [s])
            m1 = plsc.load_gather(h1, [s])
            dlo = d & 15
            dhi = jnp.right_shift(d, 4)
            plsc.addupdate_scatter(acc, [dlo, dhi], m0)
            plsc.addupdate_scatter(acc, [dlo, dhi + COL_B], m1)
            return c
        lax.fori_loop(0, ECH_W, edge, 0)
        reduce_acc()

        def post(i, c):
            sl = pl.ds(i * 16, 16)
            inn = inorm[sl]
            fi = jnp.full((16,), i, jnp.int32)
            av0 = plsc.load_gather(acc, [iota16, fi])
            av1 = plsc.load_gather(acc, [iota16, fi + COL_B])
            wr0[sl] = jnp.maximum(av0 * inn + b0, 0.0)
            wr1[sl] = jnp.maximum(av1 * inn + b1, 0.0)
            return c
        lax.fori_loop(0, NCH_N, post, 0)

    layer(lambda i: consts_v[pl.ds(OFF_F0 + i * 16, 16)],
          lambda i: consts_v[pl.ds(OFF_F1 + i * 16, 16)],
          0, x0, x1)
    layer(lambda i: x0[pl.ds(i * 16, 16)],
          lambda i: x1[pl.ds(i * 16, 16)],
          6, x0, x1)

    @pl.when(is_lead)
    def _():
        def fc(i, a):
            sl = pl.ds(i * 16, 16)
            return (a + x0[sl] * consts_v[pl.ds(OFF_W0 + i * 16, 16)]
                    + x1[sl] * consts_v[pl.ds(OFF_W1 + i * 16, 16)])
        a = lax.fori_loop(0, NCH_N, fc, zeros16)
        tot = jnp.full((16,), jnp.sum(a)) + bc(12)
        res_v[...] = 1.0 / (1.0 + jnp.exp(-tot))
        pltpu.sync_copy(res_v, out_hbm)


_gnn = functools.partial(
    pl.kernel,
    out_type=jax.ShapeDtypeStruct((16,), jnp.float32),
    mesh=plsc.VectorSubcoreMesh(core_axis_name="c", subcore_axis_name="s",
                                num_cores=1, num_subcores=16),
    compiler_params=pltpu.CompilerParams(needs_layout_passes=False),
    scratch_types=[
        pltpu.VMEM((EPT,), jnp.int32),
        pltpu.VMEM((EPT,), jnp.int32),
        pltpu.VMEM((OFF_P + 16,), jnp.float32),
        pltpu.VMEM((NPAD,), jnp.float32),
        pltpu.VMEM((NPAD,), jnp.float32),
        pltpu.VMEM((NPAD,), jnp.float32),
        pltpu.VMEM((NPAD,), jnp.float32),
        pltpu.VMEM((NPAD,), jnp.float32),
        pltpu.VMEM((NPAD,), jnp.float32),
        pltpu.VMEM((NSUB, ACC_COLS), jnp.float32),
        pltpu.VMEM_SHARED((NSUB, ACC_COLS), jnp.float32),
        pltpu.VMEM((16,), jnp.float32),
    ],
)(_gnn_body)


def kernel(features, edge_index, W1, b1, W2, b2, fc_w, fc_b):
    pad = jnp.full((EPAD - E,), N, jnp.int32)
    src = jnp.concatenate([edge_index[0], pad])
    dst = jnp.concatenate([edge_index[1], pad])

    zpad = jnp.zeros((NPAD - N,), jnp.float32)
    fcw = fc_w[:, 0].reshape(N, 2)
    consts = jnp.concatenate([
        features[:, 0], zpad,
        features[:, 1], zpad,
        fcw[:, 0], zpad,
        fcw[:, 1], zpad,
        W1.ravel(), b1, W2.ravel(), b2, fc_b, jnp.zeros((3,), jnp.float32),
    ])

    res = _gnn(src, dst, consts)
    return res[0:1].reshape(1, 1)
